# baseline (speedup 1.0000x reference)
import jax
import jax.numpy as jnp
from jax.experimental import pallas as pl
from jax.experimental.pallas import tpu as pltpu

_B = 4
_N = 4096
_MT = 1024
_LANES = 128


def _chamfer_body(a_ref, bt_ref, x2_ref, y2_ref, out_ref, acc_ref, sum_ref):
    b = pl.program_id(0)
    j = pl.program_id(1)
    nj = pl.num_programs(1)

    a = a_ref[0]
    bt = bt_ref[0]
    x2 = x2_ref[0]
    y2 = y2_ref[0]

    xy2 = jax.lax.dot_general(
        a, bt, (((1,), (0,)), ((), ())),
        preferred_element_type=jnp.float32)

    g2 = xy2 + x2
    s2 = jnp.sum(jnp.maximum(jnp.min(g2, axis=0, keepdims=True) + y2, 0.0))

    g1 = xy2 + y2
    fold = g1[:, 0:_LANES]
    for k in range(1, _MT // _LANES):
        fold = jnp.minimum(fold, g1[:, k * _LANES:(k + 1) * _LANES])

    @pl.when(j == 0)
    def _():
        acc_ref[...] = fold

    @pl.when(j > 0)
    def _():
        acc_ref[...] = jnp.minimum(acc_ref[...], fold)

    @pl.when((b == 0) & (j == 0))
    def _():
        sum_ref[0] = 0.0

    sum_ref[0] += s2

    @pl.when(j == nj - 1)
    def _():
        r1 = jnp.min(acc_ref[...], axis=1, keepdims=True)
        sum_ref[0] += jnp.sum(jnp.maximum(r1 + x2, 0.0))
        @pl.when(b == _B - 1)
        def _():
            out_ref[...] = jnp.broadcast_to(sum_ref[0], (1, 1))


@jax.jit
def kernel(points1, points2):
    x2 = jnp.sum(points1 * points1, axis=-1, keepdims=True)
    y2 = jnp.sum(points2 * points2, axis=-1, keepdims=True)
    y2t = y2.transpose(0, 2, 1)
    zeros = jnp.zeros((_B, _N, 5), jnp.float32)
    a = jnp.concatenate([-2.0 * points1, zeros], axis=-1).astype(jnp.bfloat16)
    bt = jnp.concatenate([points2, zeros], axis=-1).transpose(0, 2, 1)
    bt = bt.astype(jnp.bfloat16)

    out = pl.pallas_call(
        _chamfer_body,
        grid=(_B, _N // _MT),
        in_specs=[
            pl.BlockSpec((1, _N, 8), lambda b, j: (b, 0, 0)),
            pl.BlockSpec((1, 8, _MT), lambda b, j: (b, 0, j)),
            pl.BlockSpec((1, _N, 1), lambda b, j: (b, 0, 0)),
            pl.BlockSpec((1, 1, _MT), lambda b, j: (b, 0, j)),
        ],
        out_specs=pl.BlockSpec((1, 1), lambda b, j: (0, 0)),
        out_shape=jax.ShapeDtypeStruct((1, 1), jnp.float32),
        scratch_shapes=[
            pltpu.VMEM((_N, _LANES), jnp.float32),
            pltpu.SMEM((1,), jnp.float32),
        ],
    )(a, bt, x2, y2t)
    return out[0, 0]


# speedup vs baseline: 1.0766x; 1.0766x over previous
import jax
import jax.numpy as jnp
from jax.experimental import pallas as pl
from jax.experimental.pallas import tpu as pltpu

_B = 4
_N = 4096
_MT = 2048
_LANES = 128
_CH = 512


def _chamfer_body(a_ref, bt_ref, x2_ref, y2_ref, out_ref, acc_ref, sum_ref):
    b = pl.program_id(0)
    j = pl.program_id(1)
    nj = pl.num_programs(1)

    a = a_ref[0]
    bt = bt_ref[0]
    x2 = x2_ref[0]
    y2 = y2_ref[0]

    fold = None
    cms = []
    for c in range(_MT // _CH):
        lo, hi = c * _CH, (c + 1) * _CH
        dc = jax.lax.dot_general(
            a, bt[:, lo:hi], (((1,), (0,)), ((), ())),
            preferred_element_type=jnp.float32)
        g1 = dc + y2[:, lo:hi]
        for k in range(_CH // _LANES):
            part = g1[:, k * _LANES:(k + 1) * _LANES]
            fold = part if fold is None else jnp.minimum(fold, part)
        g2 = dc + x2
        cms.append(jnp.min(g2, axis=0, keepdims=True) + y2[:, lo:hi])

    s2 = jnp.sum(jnp.maximum(jnp.concatenate(cms, axis=1), 0.0))

    @pl.when(j == 0)
    def _():
        acc_ref[...] = fold

    @pl.when(j > 0)
    def _():
        acc_ref[...] = jnp.minimum(acc_ref[...], fold)

    @pl.when((b == 0) & (j == 0))
    def _():
        sum_ref[0] = 0.0

    sum_ref[0] += s2

    @pl.when(j == nj - 1)
    def _():
        r1 = jnp.min(acc_ref[...], axis=1, keepdims=True)
        sum_ref[0] += jnp.sum(jnp.maximum(r1 + x2, 0.0))
        @pl.when(b == _B - 1)
        def _():
            out_ref[...] = jnp.broadcast_to(sum_ref[0], (1, 1))


@jax.jit
def kernel(points1, points2):
    x2 = jnp.sum(points1 * points1, axis=-1, keepdims=True)
    y2 = jnp.sum(points2 * points2, axis=-1, keepdims=True)
    y2t = y2.transpose(0, 2, 1)
    zeros = jnp.zeros((_B, _N, 5), jnp.float32)
    a = jnp.concatenate([-2.0 * points1, zeros], axis=-1).astype(jnp.bfloat16)
    bt = jnp.concatenate([points2, zeros], axis=-1).transpose(0, 2, 1)
    bt = bt.astype(jnp.bfloat16)

    out = pl.pallas_call(
        _chamfer_body,
        grid=(_B, _N // _MT),
        in_specs=[
            pl.BlockSpec((1, _N, 8), lambda b, j: (b, 0, 0)),
            pl.BlockSpec((1, 8, _MT), lambda b, j: (b, 0, j)),
            pl.BlockSpec((1, _N, 1), lambda b, j: (b, 0, 0)),
            pl.BlockSpec((1, 1, _MT), lambda b, j: (b, 0, j)),
        ],
        out_specs=pl.BlockSpec((1, 1), lambda b, j: (0, 0)),
        out_shape=jax.ShapeDtypeStruct((1, 1), jnp.float32),
        scratch_shapes=[
            pltpu.VMEM((_N, _LANES), jnp.float32),
            pltpu.SMEM((1,), jnp.float32),
        ],
    )(a, bt, x2, y2t)
    return out[0, 0]


# speedup vs baseline: 1.1364x; 1.0555x over previous
import jax
import jax.numpy as jnp
from jax.experimental import pallas as pl
from jax.experimental.pallas import tpu as pltpu

_B = 4
_N = 4096
_M = 4096
_CH = 512
_LANES = 128


def _chamfer_body(a_ref, bt_ref, x2_ref, y2_ref, out_ref, sum_ref):
    b = pl.program_id(0)

    a = a_ref[0]
    bt = bt_ref[0]
    x2 = x2_ref[0]
    y2 = y2_ref[0]

    fold = None
    cms = []
    for c in range(_M // _CH):
        lo, hi = c * _CH, (c + 1) * _CH
        dc = jax.lax.dot_general(
            a, bt[:, lo:hi], (((1,), (0,)), ((), ())),
            preferred_element_type=jnp.float32)
        g1 = dc + y2[:, lo:hi]
        for k in range(_CH // _LANES):
            part = g1[:, k * _LANES:(k + 1) * _LANES]
            fold = part if fold is None else jnp.minimum(fold, part)
        g2 = dc + x2
        cms.append(jnp.min(g2, axis=0, keepdims=True) + y2[:, lo:hi])

    s2 = jnp.sum(jnp.maximum(jnp.concatenate(cms, axis=1), 0.0))
    r1 = jnp.min(fold, axis=1, keepdims=True)
    s1 = jnp.sum(jnp.maximum(r1 + x2, 0.0))

    @pl.when(b == 0)
    def _():
        sum_ref[0] = 0.0

    sum_ref[0] += s1 + s2

    @pl.when(b == _B - 1)
    def _():
        out_ref[...] = jnp.broadcast_to(sum_ref[0], (1, 1))


@jax.jit
def kernel(points1, points2):
    x2 = jnp.sum(points1 * points1, axis=-1, keepdims=True)
    y2 = jnp.sum(points2 * points2, axis=-1, keepdims=True)
    y2t = y2.transpose(0, 2, 1)
    zeros = jnp.zeros((_B, _N, 5), jnp.float32)
    a = jnp.concatenate([-2.0 * points1, zeros], axis=-1).astype(jnp.bfloat16)
    bt = jnp.concatenate([points2, zeros], axis=-1).transpose(0, 2, 1)
    bt = bt.astype(jnp.bfloat16)

    out = pl.pallas_call(
        _chamfer_body,
        grid=(_B,),
        in_specs=[
            pl.BlockSpec((1, _N, 8), lambda b: (b, 0, 0)),
            pl.BlockSpec((1, 8, _M), lambda b: (b, 0, 0)),
            pl.BlockSpec((1, _N, 1), lambda b: (b, 0, 0)),
            pl.BlockSpec((1, 1, _M), lambda b: (b, 0, 0)),
        ],
        out_specs=pl.BlockSpec((1, 1), lambda b: (0, 0)),
        out_shape=jax.ShapeDtypeStruct((1, 1), jnp.float32),
        scratch_shapes=[
            pltpu.SMEM((1,), jnp.float32),
        ],
    )(a, bt, x2, y2t)
    return out[0, 0]
